# baseline (speedup 1.0000x reference)
import functools

import jax
import jax.numpy as jnp
from jax import lax
from jax.experimental import pallas as pl
from jax.experimental.pallas import tpu as pltpu
from jax.experimental.pallas import tpu_sc as plsc

NUM_GENES = 100000
NUM_SPOTS = 1000000
LATENT_DIM = 32
BATCH = 16384

_NC = 2
_NS = 16
_L = 16
_NW = _NC * _NS
_BPW = BATCH // _NW


def _nmf_body(gidx_hbm, sidx_hbm, gtab_hbm, stab_hbm, out_hbm,
              gidx_v, sidx_v, grows_v, srows_v, out_v, sem_g, sem_s):
    wid = lax.axis_index("s") * _NC + lax.axis_index("c")
    base = wid * _BPW

    pltpu.sync_copy(gidx_hbm.at[pl.ds(base, _BPW)], gidx_v)
    pltpu.sync_copy(sidx_hbm.at[pl.ds(base, _BPW)], sidx_v)

    copies = []
    for d in range(LATENT_DIM):
        copies.append(pltpu.async_copy(
            gtab_hbm.at[d].at[gidx_v], grows_v.at[d], sem_g))
        copies.append(pltpu.async_copy(
            stab_hbm.at[d].at[sidx_v], srows_v.at[d], sem_s))
    for c in copies:
        c.wait()

    def chunk(c, carry):
        cb = c * _L
        acc = jnp.zeros((_L,), jnp.float32)
        for d in range(LATENT_DIM):
            acc = acc + grows_v[d, pl.ds(cb, _L)] * srows_v[d, pl.ds(cb, _L)]
        out_v[pl.ds(cb, _L)] = acc
        return carry

    lax.fori_loop(0, _BPW // _L, chunk, 0)
    pltpu.sync_copy(out_v, out_hbm.at[pl.ds(base, _BPW)])


@jax.jit
def _nmf_sc(gene_indices, spot_indices, embedding_genes, embedding_spots):
    mesh = plsc.VectorSubcoreMesh(core_axis_name="c", subcore_axis_name="s")
    run = functools.partial(
        pl.kernel,
        out_type=jax.ShapeDtypeStruct((BATCH,), jnp.float32),
        mesh=mesh,
        compiler_params=pltpu.CompilerParams(
            use_tc_tiling_on_sc=False, needs_layout_passes=False),
        scratch_types=[
            pltpu.VMEM((_BPW,), jnp.int32),
            pltpu.VMEM((_BPW,), jnp.int32),
            pltpu.VMEM((LATENT_DIM, _BPW), jnp.float32),
            pltpu.VMEM((LATENT_DIM, _BPW), jnp.float32),
            pltpu.VMEM((_BPW,), jnp.float32),
            pltpu.SemaphoreType.DMA,
            pltpu.SemaphoreType.DMA,
        ],
    )(_nmf_body)
    return run(gene_indices, spot_indices,
               embedding_genes.T, embedding_spots.T)


def kernel(gene_indices, spot_indices, embedding_genes, embedding_spots):
    gene_indices = gene_indices.astype(jnp.int32)
    spot_indices = spot_indices.astype(jnp.int32)
    return _nmf_sc(gene_indices, spot_indices, embedding_genes,
                   embedding_spots)


# speedup vs baseline: 16.0960x; 16.0960x over previous
import functools

import jax
import jax.numpy as jnp
from jax import lax
from jax.experimental import pallas as pl
from jax.experimental.pallas import tpu as pltpu
from jax.experimental.pallas import tpu_sc as plsc

NUM_GENES = 100000
NUM_SPOTS = 1000000
LATENT_DIM = 32
BATCH = 16384

_NC = 2
_NS = 16
_L = 16
_NW = _NC * _NS
_BPW = BATCH // _NW

_S_STRIDE = 1000008
_G_STRIDE = 100008
_S_WCOLS = 244 * 128
_S_CHUNK = 61 * 128
_S_EDGE_OFF = 7808 * 128
_S_EDGE_LEN = 512
_S_TAIL_OFF = 7812 * 128
_S_TAIL_LEN = 64
_G_CHUNK = 24 * 128
_G_EDGE_OFF = 768 * 128
_G_EDGE_LEN = 1664
_G_TAIL_OFF = 781 * 128
_G_TAIL_LEN = 32
_NBUF = 4


def _detile_body(gtab_hbm, stab_hbm, gflat_hbm, sflat_hbm,
                 b0, b1, b2, b3, f0, f1, f2, f3, w0, w1, w2, w3):
    wid = lax.axis_index("s") * _NC + lax.axis_index("c")
    bufs = (b0, b1, b2, b3)
    fsems = (f0, f1, f2, f3)
    wsems = (w0, w1, w2, w3)

    seq = []
    for d in range(LATENT_DIM):
        for k in range(4):
            seq.append(("s", d, k, _S_CHUNK))
    for d in range(LATENT_DIM):
        seq.append(("g", d, 0, _G_CHUNK))
    seq.append(("se", 0, 0, _S_EDGE_LEN))
    seq.append(("ge", 0, 0, _G_EDGE_LEN))
    seq.append(("sp", 0, 0, _S_TAIL_LEN))
    seq.append(("gp", 0, 0, _G_TAIL_LEN))
    n = len(seq)

    def refs(i):
        which, d, k, ln = seq[i]
        if which == "s":
            off = pl.multiple_of(wid * _S_WCOLS + k * _S_CHUNK, 128)
            src = stab_hbm.at[d].at[pl.ds(off, ln)]
            dst = sflat_hbm.at[pl.ds(
                pl.multiple_of(d * _S_STRIDE + wid * _S_WCOLS
                               + k * _S_CHUNK, 8), ln)]
        elif which == "g":
            off = pl.multiple_of(wid * _G_CHUNK, 128)
            src = gtab_hbm.at[d].at[pl.ds(off, ln)]
            dst = gflat_hbm.at[pl.ds(
                pl.multiple_of(d * _G_STRIDE + wid * _G_CHUNK, 8), ln)]
        elif which == "se":
            src = stab_hbm.at[wid].at[pl.ds(_S_EDGE_OFF, ln)]
            dst = sflat_hbm.at[pl.ds(
                pl.multiple_of(wid * _S_STRIDE + _S_EDGE_OFF, 8), ln)]
        elif which == "ge":
            src = gtab_hbm.at[wid].at[pl.ds(_G_EDGE_OFF, ln)]
            dst = gflat_hbm.at[pl.ds(
                pl.multiple_of(wid * _G_STRIDE + _G_EDGE_OFF, 8), ln)]
        elif which == "sp":
            src = stab_hbm.at[wid].at[pl.ds(_S_TAIL_OFF, ln)]
            dst = sflat_hbm.at[pl.ds(
                pl.multiple_of(wid * _S_STRIDE + _S_TAIL_OFF, 8), ln)]
        else:
            src = gtab_hbm.at[wid].at[pl.ds(_G_TAIL_OFF, ln)]
            dst = gflat_hbm.at[pl.ds(
                pl.multiple_of(wid * _G_STRIDE + _G_TAIL_OFF, 8), ln)]
        return src, dst, ln, ln

    def issue_fetch(i):
        src, _, fln, _ln = refs(i)
        pltpu.async_copy(src, bufs[i % _NBUF].at[pl.ds(0, fln)],
                         fsems[i % _NBUF])

    for i in range(min(_NBUF - 1, n)):
        issue_fetch(i)
    for i in range(n):
        src, dst, fln, ln = refs(i)
        pltpu.make_async_copy(src, bufs[i % _NBUF].at[pl.ds(0, fln)],
                              fsems[i % _NBUF]).wait()
        pltpu.async_copy(bufs[i % _NBUF].at[pl.ds(0, ln)], dst,
                         wsems[i % _NBUF])
        j = i + _NBUF - 1
        if j < n:
            p = j - _NBUF
            if p >= 0:
                _ps, pdst, _pf, pln = refs(p)
                pltpu.make_async_copy(
                    bufs[p % _NBUF].at[pl.ds(0, pln)], pdst,
                    wsems[p % _NBUF]).wait()
            issue_fetch(j)
    for i in range(max(0, n - _NBUF), n):
        _s2, dst, _f2, ln = refs(i)
        pltpu.make_async_copy(bufs[i % _NBUF].at[pl.ds(0, ln)], dst,
                              wsems[i % _NBUF]).wait()


def _gather_body(gidx_hbm, sidx_hbm, gflat_hbm, sflat_hbm, out_hbm,
                 gidx_v, sidx_v, grows_v, srows_v, out_v, sem_g, sem_s):
    wid = lax.axis_index("s") * _NC + lax.axis_index("c")
    base = wid * _BPW

    pltpu.sync_copy(gidx_hbm.at[pl.ds(base, _BPW)], gidx_v)
    pltpu.sync_copy(sidx_hbm.at[pl.ds(base, _BPW)], sidx_v)

    copies = []
    for d in range(LATENT_DIM):
        copies.append(pltpu.async_copy(
            gflat_hbm.at[pl.ds(d * _G_STRIDE, _G_STRIDE)].at[gidx_v],
            grows_v.at[d], sem_g))
        copies.append(pltpu.async_copy(
            sflat_hbm.at[pl.ds(d * _S_STRIDE, _S_STRIDE)].at[sidx_v],
            srows_v.at[d], sem_s))
    for c in copies:
        c.wait()

    def chunk(c, carry):
        cb = c * _L
        acc = jnp.zeros((_L,), jnp.float32)
        for d in range(LATENT_DIM):
            acc = acc + grows_v[d, pl.ds(cb, _L)] * srows_v[d, pl.ds(cb, _L)]
        out_v[pl.ds(cb, _L)] = acc
        return carry

    lax.fori_loop(0, _BPW // _L, chunk, 0)
    pltpu.sync_copy(out_v, out_hbm.at[pl.ds(base, _BPW)])


@jax.jit
def _nmf_sc(gene_indices, spot_indices, embedding_genes, embedding_spots):
    mesh = plsc.VectorSubcoreMesh(core_axis_name="c", subcore_axis_name="s")

    detile = functools.partial(
        pl.kernel,
        out_type=(jax.ShapeDtypeStruct((LATENT_DIM * _G_STRIDE,),
                                       jnp.float32),
                  jax.ShapeDtypeStruct((LATENT_DIM * _S_STRIDE,),
                                       jnp.float32)),
        mesh=mesh,
        compiler_params=pltpu.CompilerParams(needs_layout_passes=False),
        scratch_types=(
            [pltpu.VMEM((_S_CHUNK,), jnp.float32) for _ in range(_NBUF)]
            + [pltpu.SemaphoreType.DMA] * (2 * _NBUF)
        ),
    )(_detile_body)
    gflat, sflat = detile(embedding_genes.T, embedding_spots.T)

    gather = functools.partial(
        pl.kernel,
        out_type=jax.ShapeDtypeStruct((BATCH,), jnp.float32),
        mesh=mesh,
        compiler_params=pltpu.CompilerParams(
            use_tc_tiling_on_sc=False, needs_layout_passes=False),
        scratch_types=[
            pltpu.VMEM((_BPW,), jnp.int32),
            pltpu.VMEM((_BPW,), jnp.int32),
            pltpu.VMEM((LATENT_DIM, _BPW), jnp.float32),
            pltpu.VMEM((LATENT_DIM, _BPW), jnp.float32),
            pltpu.VMEM((_BPW,), jnp.float32),
            pltpu.SemaphoreType.DMA,
            pltpu.SemaphoreType.DMA,
        ],
    )(_gather_body)
    return gather(gene_indices, spot_indices, gflat, sflat)


def kernel(gene_indices, spot_indices, embedding_genes, embedding_spots):
    gene_indices = gene_indices.astype(jnp.int32)
    spot_indices = spot_indices.astype(jnp.int32)
    return _nmf_sc(gene_indices, spot_indices, embedding_genes,
                   embedding_spots)


# speedup vs baseline: 16.2411x; 1.0090x over previous
import functools

import jax
import jax.numpy as jnp
from jax import lax
from jax.experimental import pallas as pl
from jax.experimental.pallas import tpu as pltpu
from jax.experimental.pallas import tpu_sc as plsc

NUM_GENES = 100000
NUM_SPOTS = 1000000
LATENT_DIM = 32
BATCH = 16384

_NC = 2
_NS = 16
_L = 16
_NW = _NC * _NS
_BPW = BATCH // _NW

_S_STRIDE = 1000008
_G_STRIDE = 100008
_S_WCOLS = 244 * 128
_S_CHUNK = 122 * 128
_S_EDGE_OFF = 7808 * 128
_S_EDGE_LEN = 512
_S_TAIL_OFF = 7812 * 128
_S_TAIL_LEN = 64
_G_CHUNK = 24 * 128
_G_EDGE_OFF = 768 * 128
_G_EDGE_LEN = 1664
_G_TAIL_OFF = 781 * 128
_G_TAIL_LEN = 32
_NBUF = 6


def _detile_body(gtab_hbm, stab_hbm, gflat_hbm, sflat_hbm,
                 b0, b1, b2, b3, b4, b5, f0, f1, f2, f3, f4, f5,
                 w0, w1, w2, w3, w4, w5):
    wid = lax.axis_index("s") * _NC + lax.axis_index("c")
    bufs = (b0, b1, b2, b3, b4, b5)
    fsems = (f0, f1, f2, f3, f4, f5)
    wsems = (w0, w1, w2, w3, w4, w5)

    seq = []
    for d in range(LATENT_DIM):
        for k in range(2):
            seq.append(("s", d, k, _S_CHUNK))
    for d in range(LATENT_DIM):
        seq.append(("g", d, 0, _G_CHUNK))
    seq.append(("se", 0, 0, _S_EDGE_LEN))
    seq.append(("ge", 0, 0, _G_EDGE_LEN))
    seq.append(("sp", 0, 0, _S_TAIL_LEN))
    seq.append(("gp", 0, 0, _G_TAIL_LEN))
    n = len(seq)

    def refs(i):
        which, d, k, ln = seq[i]
        if which == "s":
            off = pl.multiple_of(wid * _S_WCOLS + k * _S_CHUNK, 128)
            src = stab_hbm.at[d].at[pl.ds(off, ln)]
            dst = sflat_hbm.at[pl.ds(
                pl.multiple_of(d * _S_STRIDE + wid * _S_WCOLS
                               + k * _S_CHUNK, 8), ln)]
        elif which == "g":
            off = pl.multiple_of(wid * _G_CHUNK, 128)
            src = gtab_hbm.at[d].at[pl.ds(off, ln)]
            dst = gflat_hbm.at[pl.ds(
                pl.multiple_of(d * _G_STRIDE + wid * _G_CHUNK, 8), ln)]
        elif which == "se":
            src = stab_hbm.at[wid].at[pl.ds(_S_EDGE_OFF, ln)]
            dst = sflat_hbm.at[pl.ds(
                pl.multiple_of(wid * _S_STRIDE + _S_EDGE_OFF, 8), ln)]
        elif which == "ge":
            src = gtab_hbm.at[wid].at[pl.ds(_G_EDGE_OFF, ln)]
            dst = gflat_hbm.at[pl.ds(
                pl.multiple_of(wid * _G_STRIDE + _G_EDGE_OFF, 8), ln)]
        elif which == "sp":
            src = stab_hbm.at[wid].at[pl.ds(_S_TAIL_OFF, ln)]
            dst = sflat_hbm.at[pl.ds(
                pl.multiple_of(wid * _S_STRIDE + _S_TAIL_OFF, 8), ln)]
        else:
            src = gtab_hbm.at[wid].at[pl.ds(_G_TAIL_OFF, ln)]
            dst = gflat_hbm.at[pl.ds(
                pl.multiple_of(wid * _G_STRIDE + _G_TAIL_OFF, 8), ln)]
        return src, dst, ln, ln

    def issue_fetch(i):
        src, _, fln, _ln = refs(i)
        pltpu.async_copy(src, bufs[i % _NBUF].at[pl.ds(0, fln)],
                         fsems[i % _NBUF])

    for i in range(min(_NBUF - 1, n)):
        issue_fetch(i)
    for i in range(n):
        src, dst, fln, ln = refs(i)
        pltpu.make_async_copy(src, bufs[i % _NBUF].at[pl.ds(0, fln)],
                              fsems[i % _NBUF]).wait()
        pltpu.async_copy(bufs[i % _NBUF].at[pl.ds(0, ln)], dst,
                         wsems[i % _NBUF])
        j = i + _NBUF - 1
        if j < n:
            p = j - _NBUF
            if p >= 0:
                _ps, pdst, _pf, pln = refs(p)
                pltpu.make_async_copy(
                    bufs[p % _NBUF].at[pl.ds(0, pln)], pdst,
                    wsems[p % _NBUF]).wait()
            issue_fetch(j)
    for i in range(max(0, n - _NBUF), n):
        _s2, dst, _f2, ln = refs(i)
        pltpu.make_async_copy(bufs[i % _NBUF].at[pl.ds(0, ln)], dst,
                              wsems[i % _NBUF]).wait()


def _gather_body(gidx_hbm, sidx_hbm, gflat_hbm, sflat_hbm, out_hbm,
                 gidx_v, sidx_v, grows_v, srows_v, out_v, sem_g, sem_s):
    wid = lax.axis_index("s") * _NC + lax.axis_index("c")
    base = wid * _BPW

    pltpu.sync_copy(gidx_hbm.at[pl.ds(base, _BPW)], gidx_v)
    pltpu.sync_copy(sidx_hbm.at[pl.ds(base, _BPW)], sidx_v)

    copies = []
    for d in range(LATENT_DIM):
        copies.append(pltpu.async_copy(
            gflat_hbm.at[pl.ds(d * _G_STRIDE, _G_STRIDE)].at[gidx_v],
            grows_v.at[d], sem_g))
        copies.append(pltpu.async_copy(
            sflat_hbm.at[pl.ds(d * _S_STRIDE, _S_STRIDE)].at[sidx_v],
            srows_v.at[d], sem_s))
    for c in copies:
        c.wait()

    def chunk(c, carry):
        cb = c * _L
        acc = jnp.zeros((_L,), jnp.float32)
        for d in range(LATENT_DIM):
            acc = acc + grows_v[d, pl.ds(cb, _L)] * srows_v[d, pl.ds(cb, _L)]
        out_v[pl.ds(cb, _L)] = acc
        return carry

    lax.fori_loop(0, _BPW // _L, chunk, 0)
    pltpu.sync_copy(out_v, out_hbm.at[pl.ds(base, _BPW)])


@jax.jit
def _nmf_sc(gene_indices, spot_indices, embedding_genes, embedding_spots):
    mesh = plsc.VectorSubcoreMesh(core_axis_name="c", subcore_axis_name="s")

    detile = functools.partial(
        pl.kernel,
        out_type=(jax.ShapeDtypeStruct((LATENT_DIM * _G_STRIDE,),
                                       jnp.float32),
                  jax.ShapeDtypeStruct((LATENT_DIM * _S_STRIDE,),
                                       jnp.float32)),
        mesh=mesh,
        compiler_params=pltpu.CompilerParams(needs_layout_passes=False),
        scratch_types=(
            [pltpu.VMEM((_S_CHUNK,), jnp.float32) for _ in range(_NBUF)]
            + [pltpu.SemaphoreType.DMA] * (2 * _NBUF)
        ),
    )(_detile_body)
    gflat, sflat = detile(embedding_genes.T, embedding_spots.T)

    gather = functools.partial(
        pl.kernel,
        out_type=jax.ShapeDtypeStruct((BATCH,), jnp.float32),
        mesh=mesh,
        compiler_params=pltpu.CompilerParams(
            use_tc_tiling_on_sc=False, needs_layout_passes=False),
        scratch_types=[
            pltpu.VMEM((_BPW,), jnp.int32),
            pltpu.VMEM((_BPW,), jnp.int32),
            pltpu.VMEM((LATENT_DIM, _BPW), jnp.float32),
            pltpu.VMEM((LATENT_DIM, _BPW), jnp.float32),
            pltpu.VMEM((_BPW,), jnp.float32),
            pltpu.SemaphoreType.DMA,
            pltpu.SemaphoreType.DMA,
        ],
    )(_gather_body)
    return gather(gene_indices, spot_indices, gflat, sflat)


def kernel(gene_indices, spot_indices, embedding_genes, embedding_spots):
    gene_indices = gene_indices.astype(jnp.int32)
    spot_indices = spot_indices.astype(jnp.int32)
    return _nmf_sc(gene_indices, spot_indices, embedding_genes,
                   embedding_spots)
